# baseline (speedup 1.0000x reference)
import functools

import jax
import jax.numpy as jnp
from jax.experimental import pallas as pl
from jax.experimental.pallas import tpu as pltpu
from jax.experimental.pallas import tpu_sc as plsc

_EPS = 1.1920928955078125e-07


def _dot_t(a, b):
    return jax.lax.dot_general(a, b, (((1,), (1,)), ((), ())),
                               preferred_element_type=jnp.float32)


def _rms(x, w):
    ms = jnp.mean(x * x, axis=-1, keepdims=True)
    return x / jnp.sqrt(ms + _EPS) * w


def _qkv_kernel(x_ref, n1_ref, wq_ref, wk_ref, wv_ref, q_ref, k_ref, v_ref):
    xn = _rms(x_ref[...], n1_ref[...])
    q_ref[...] = _dot_t(xn, wq_ref[...])
    k_ref[...] = _dot_t(xn, wk_ref[...])
    v_ref[...] = _dot_t(xn, wv_ref[...])


def _attn_kernel(q_ref, k_ref, v_ref, o_ref, *, scale, bq):
    i = pl.program_id(1)
    q = q_ref[0]
    k = k_ref[0]
    v = v_ref[0]
    sc = _dot_t(q, k) * scale
    row = jax.lax.broadcasted_iota(jnp.int32, sc.shape, 0) + i * bq
    col = jax.lax.broadcasted_iota(jnp.int32, sc.shape, 1)
    sc = sc + jnp.where(col <= row, 1.0, -10000.0)
    m = jnp.max(sc, axis=-1, keepdims=True)
    p = jnp.exp(sc - m)
    p = p / jnp.sum(p, axis=-1, keepdims=True)
    o_ref[0] = jnp.dot(p, v, preferred_element_type=jnp.float32)


def _proj_kernel(o_ref, res_ref, w_ref, n2_ref, x2_ref, h2_ref):
    x2 = _dot_t(o_ref[...], w_ref[...]) + res_ref[...]
    x2_ref[...] = x2
    h2_ref[...] = _rms(x2, n2_ref[...])


def _gate_kernel(h2_ref, gw_ref, i1_ref, i2_ref, w1_ref, w2_ref, loss_ref,
                 acc_ref, *, n_total, n_blocks):
    i = pl.program_id(0)
    g = _dot_t(h2_ref[...], gw_ref[...])
    m = jnp.max(g, axis=-1, keepdims=True)
    eg = jnp.exp(g - m)
    gate = eg / jnp.sum(eg, axis=-1, keepdims=True)
    i1 = jnp.argmax(gate, axis=-1)
    w1 = jnp.max(gate, axis=-1)
    lane = jax.lax.broadcasted_iota(jnp.int32, gate.shape, 1)
    oh1 = lane == i1[:, None]
    g2 = jnp.where(oh1, -jnp.inf, gate)
    i2 = jnp.argmax(g2, axis=-1)
    w2 = jnp.max(g2, axis=-1)
    i1_ref[...] = i1[:, None].astype(jnp.int32)
    i2_ref[...] = i2[:, None].astype(jnp.int32)
    w1_ref[...] = w1[:, None]
    w2_ref[...] = w2[:, None]
    psum = jnp.sum(gate, axis=0, keepdims=True)

    @pl.when(i == 0)
    def _():
        acc_ref[...] = psum

    @pl.when(i > 0)
    def _():
        acc_ref[...] += psum

    @pl.when(i == n_blocks - 1)
    def _():
        p = acc_ref[...] / n_total
        mu = jnp.mean(p)
        var = jnp.mean((p - mu) ** 2)
        loss_ref[...] = (var / (mu * mu + 1e-10)).reshape(1, 1)


def _ffn_kernel(be_ref, xg_ref, w1_ref, w2_ref, ws_ref, yp_ref):
    del be_ref
    h = _dot_t(xg_ref[...], w1_ref[0])
    h = h * jax.nn.sigmoid(h)
    y = _dot_t(h, w2_ref[0])
    yp_ref[...] = y * ws_ref[...]


def _comb_kernel(x2_ref, g_ref, out_ref):
    out_ref[...] = x2_ref[...] + g_ref[:, 0] + g_ref[:, 1]


def _sc_gather(table, idx, n_rows, n_cols, chunk):
    info = plsc.get_sparse_core_info()
    nc = info.num_cores
    nw = nc * info.num_subcores
    rpw = n_rows // nw
    nchunks = rpw // chunk
    mesh = plsc.VectorSubcoreMesh(core_axis_name="c", subcore_axis_name="s")

    @functools.partial(
        pl.kernel,
        out_type=jax.ShapeDtypeStruct((n_rows, n_cols), jnp.float32),
        mesh=mesh,
        scratch_types=[
            pltpu.VMEM((rpw,), jnp.int32),
            pltpu.VMEM((chunk, n_cols), jnp.float32),
            pltpu.VMEM((chunk, n_cols), jnp.float32),
            pltpu.SemaphoreType.DMA,
            pltpu.SemaphoreType.DMA,
            pltpu.SemaphoreType.DMA,
            pltpu.SemaphoreType.DMA,
        ],
    )
    def gk(table_hbm, idx_hbm, out_hbm, idx_v, rows_a, rows_b, gs_a, gs_b,
           os_a, os_b):
        wid = jax.lax.axis_index("s") * nc + jax.lax.axis_index("c")
        base = wid * rpw
        pltpu.sync_copy(idx_hbm.at[pl.ds(base, rpw)], idx_v)
        bufs = (rows_a, rows_b)
        gsems = (gs_a, gs_b)
        osems = (os_a, os_b)

        def gstart(c):
            b = c % 2
            return pltpu.async_copy(
                table_hbm.at[idx_v.at[pl.ds(c * chunk, chunk)]], bufs[b],
                gsems[b])

        def ostart(c):
            b = c % 2
            return pltpu.async_copy(
                bufs[b], out_hbm.at[pl.ds(base + c * chunk, chunk)], osems[b])

        gh = [None] * nchunks
        oh = [None] * nchunks
        gh[0] = gstart(0)
        for c in range(nchunks):
            gh[c].wait()
            if c + 1 < nchunks:
                if c >= 1:
                    oh[c - 1].wait()
                gh[c + 1] = gstart(c + 1)
            oh[c] = ostart(c)
        oh[nchunks - 1].wait()
        if nchunks >= 2:
            oh[nchunks - 2].wait()

    return gk(table, idx)


def kernel(x, attention_mask, norm1_w, qkv_w, out_w, norm2_w, gating_w, W1, W2):
    B, S, E = x.shape
    N = B * S
    H = 16
    D = E // H
    NE, FH, _ = W1.shape

    BM = 256
    BQ = 256
    BN = 512
    BT = 256
    N2 = 2 * N
    P = N2 + NE * BT
    NB = P // BT
    f32 = jnp.float32

    xf = x.reshape(N, E)
    n1 = norm1_w.reshape(1, E)
    n2 = norm2_w.reshape(1, E)

    nj = E // BN
    ni = N // BM
    q, k, v = pl.pallas_call(
        _qkv_kernel,
        grid=(nj, ni),
        in_specs=[
            pl.BlockSpec((BM, E), lambda j, i: (i, 0)),
            pl.BlockSpec((1, E), lambda j, i: (0, 0)),
            pl.BlockSpec((BN, E), lambda j, i: (j, 0)),
            pl.BlockSpec((BN, E), lambda j, i: (j + nj, 0)),
            pl.BlockSpec((BN, E), lambda j, i: (j + 2 * nj, 0)),
        ],
        out_specs=[
            pl.BlockSpec((BM, BN), lambda j, i: (i, j)),
            pl.BlockSpec((BM, BN), lambda j, i: (i, j)),
            pl.BlockSpec((BM, BN), lambda j, i: (i, j)),
        ],
        out_shape=[jax.ShapeDtypeStruct((N, E), f32)] * 3,
    )(xf, n1, qkv_w, qkv_w, qkv_w)

    q3 = q.reshape(B * H, S, D)
    k3 = k.reshape(B * H, S, D)
    v3 = v.reshape(B * H, S, D)

    o = pl.pallas_call(
        functools.partial(_attn_kernel, scale=1.0 / (D ** 0.5), bq=BQ),
        grid=(B * H, S // BQ),
        in_specs=[
            pl.BlockSpec((1, BQ, D), lambda b, i: (b, i, 0)),
            pl.BlockSpec((1, S, D), lambda b, i: (b, 0, 0)),
            pl.BlockSpec((1, S, D), lambda b, i: (b, 0, 0)),
        ],
        out_specs=pl.BlockSpec((1, BQ, D), lambda b, i: (b, i, 0)),
        out_shape=jax.ShapeDtypeStruct((B * H, S, D), f32),
    )(q3, k3, v3)
    o2 = o.reshape(N, E)

    x2, h2 = pl.pallas_call(
        _proj_kernel,
        grid=(ni,),
        in_specs=[
            pl.BlockSpec((BM, E), lambda i: (i, 0)),
            pl.BlockSpec((BM, E), lambda i: (i, 0)),
            pl.BlockSpec((E, E), lambda i: (0, 0)),
            pl.BlockSpec((1, E), lambda i: (0, 0)),
        ],
        out_specs=[
            pl.BlockSpec((BM, E), lambda i: (i, 0)),
            pl.BlockSpec((BM, E), lambda i: (i, 0)),
        ],
        out_shape=[jax.ShapeDtypeStruct((N, E), f32)] * 2,
    )(o2, xf, out_w, n2)

    i1, i2, w1, w2, loss = pl.pallas_call(
        functools.partial(_gate_kernel, n_total=float(N), n_blocks=ni),
        grid=(ni,),
        in_specs=[
            pl.BlockSpec((BM, E), lambda i: (i, 0)),
            pl.BlockSpec((NE, E), lambda i: (0, 0)),
        ],
        out_specs=[
            pl.BlockSpec((BM, 1), lambda i: (i, 0)),
            pl.BlockSpec((BM, 1), lambda i: (i, 0)),
            pl.BlockSpec((BM, 1), lambda i: (i, 0)),
            pl.BlockSpec((BM, 1), lambda i: (i, 0)),
            pl.BlockSpec((1, 1), lambda i: (0, 0)),
        ],
        out_shape=[
            jax.ShapeDtypeStruct((N, 1), jnp.int32),
            jax.ShapeDtypeStruct((N, 1), jnp.int32),
            jax.ShapeDtypeStruct((N, 1), f32),
            jax.ShapeDtypeStruct((N, 1), f32),
            jax.ShapeDtypeStruct((1, 1), f32),
        ],
        scratch_shapes=[pltpu.VMEM((1, NE), f32)],
    )(h2, gating_w)

    e_flat = jnp.concatenate([i1, i2], axis=1).reshape(N2)
    w_flat = jnp.concatenate([w1, w2], axis=1).reshape(N2)
    oh = (e_flat[:, None] == jnp.arange(NE, dtype=jnp.int32)).astype(jnp.int32)
    csum = jnp.cumsum(oh, axis=0)
    counts = csum[-1]
    rank = jnp.take_along_axis(csum, e_flat[:, None], axis=1)[:, 0] - 1
    nblk = (counts + BT - 1) // BT
    poff = BT * jnp.concatenate(
        [jnp.zeros((1,), jnp.int32), jnp.cumsum(nblk)]).astype(jnp.int32)
    pos = poff[e_flat] + rank
    tok = jnp.zeros((P,), jnp.int32).at[pos].set(
        jnp.arange(N2, dtype=jnp.int32) // 2)
    wslot = jnp.zeros((P, 1), f32).at[pos].set(w_flat[:, None])
    block_expert = jnp.clip(
        jnp.searchsorted(poff[1:], jnp.arange(NB, dtype=jnp.int32) * BT,
                         side="right"),
        0, NE - 1).astype(jnp.int32)

    xg = _sc_gather(h2, tok, P, E, chunk=16)

    yp = pl.pallas_call(
        _ffn_kernel,
        grid_spec=pltpu.PrefetchScalarGridSpec(
            num_scalar_prefetch=1,
            grid=(NB,),
            in_specs=[
                pl.BlockSpec((BT, E), lambda b, be: (b, 0)),
                pl.BlockSpec((1, FH, E), lambda b, be: (be[b], 0, 0)),
                pl.BlockSpec((1, E, FH), lambda b, be: (be[b], 0, 0)),
                pl.BlockSpec((BT, 1), lambda b, be: (b, 0)),
            ],
            out_specs=pl.BlockSpec((BT, E), lambda b, be: (b, 0)),
        ),
        out_shape=jax.ShapeDtypeStruct((P, E), f32),
    )(block_expert, xg, W1, W2, wslot)

    g = _sc_gather(yp, pos, N2, E, chunk=16)
    g3 = g.reshape(N, 2, E)

    out = pl.pallas_call(
        _comb_kernel,
        grid=(ni,),
        in_specs=[
            pl.BlockSpec((BM, E), lambda i: (i, 0)),
            pl.BlockSpec((BM, 2, E), lambda i: (i, 0, 0)),
        ],
        out_specs=pl.BlockSpec((BM, E), lambda i: (i, 0)),
        out_shape=jax.ShapeDtypeStruct((N, E), f32),
    )(x2, g3)

    return out.reshape(B, S, E), loss[0, 0]


# speedup vs baseline: 1.0906x; 1.0906x over previous
import functools

import jax
import jax.numpy as jnp
from jax.experimental import pallas as pl
from jax.experimental.pallas import tpu as pltpu
from jax.experimental.pallas import tpu_sc as plsc

_EPS = 1.1920928955078125e-07


def _dot_t(a, b):
    return jax.lax.dot_general(a, b, (((1,), (1,)), ((), ())),
                               preferred_element_type=jnp.float32)


def _rms(x, w):
    ms = jnp.mean(x * x, axis=-1, keepdims=True)
    return x / jnp.sqrt(ms + _EPS) * w


def _qkv_kernel(x_ref, n1_ref, wq_ref, wk_ref, wv_ref, q_ref, k_ref, v_ref):
    xn = _rms(x_ref[...], n1_ref[...])
    q_ref[...] = _dot_t(xn, wq_ref[...])
    k_ref[...] = _dot_t(xn, wk_ref[...])
    v_ref[...] = _dot_t(xn, wv_ref[...])


def _attn_kernel(q_ref, k_ref, v_ref, o_ref, *, scale, bq):
    i = pl.program_id(1)
    q = q_ref[0]
    k = k_ref[0]
    v = v_ref[0]
    sc = _dot_t(q, k) * scale
    row = jax.lax.broadcasted_iota(jnp.int32, sc.shape, 0) + i * bq
    col = jax.lax.broadcasted_iota(jnp.int32, sc.shape, 1)
    sc = sc + jnp.where(col <= row, 1.0, -10000.0)
    m = jnp.max(sc, axis=-1, keepdims=True)
    p = jnp.exp(sc - m)
    p = p / jnp.sum(p, axis=-1, keepdims=True)
    o_ref[0] = jnp.dot(p, v, preferred_element_type=jnp.float32)


def _proj_kernel(o_ref, res_ref, w_ref, n2_ref, x2_ref, h2_ref):
    x2 = _dot_t(o_ref[...], w_ref[...]) + res_ref[...]
    x2_ref[...] = x2
    h2_ref[...] = _rms(x2, n2_ref[...])


def _gate_kernel(h2_ref, gw_ref, i1_ref, i2_ref, w1_ref, w2_ref, loss_ref,
                 acc_ref, *, n_total, n_blocks):
    i = pl.program_id(0)
    g = _dot_t(h2_ref[...], gw_ref[...])
    m = jnp.max(g, axis=-1, keepdims=True)
    eg = jnp.exp(g - m)
    gate = eg / jnp.sum(eg, axis=-1, keepdims=True)
    i1 = jnp.argmax(gate, axis=-1)
    w1 = jnp.max(gate, axis=-1)
    lane = jax.lax.broadcasted_iota(jnp.int32, gate.shape, 1)
    oh1 = lane == i1[:, None]
    g2 = jnp.where(oh1, -jnp.inf, gate)
    i2 = jnp.argmax(g2, axis=-1)
    w2 = jnp.max(g2, axis=-1)
    i1_ref[...] = i1[:, None].astype(jnp.int32)
    i2_ref[...] = i2[:, None].astype(jnp.int32)
    w1_ref[...] = w1[:, None]
    w2_ref[...] = w2[:, None]
    psum = jnp.sum(gate, axis=0, keepdims=True)

    @pl.when(i == 0)
    def _():
        acc_ref[...] = psum

    @pl.when(i > 0)
    def _():
        acc_ref[...] += psum

    @pl.when(i == n_blocks - 1)
    def _():
        p = acc_ref[...] / n_total
        mu = jnp.mean(p)
        var = jnp.mean((p - mu) ** 2)
        loss_ref[...] = (var / (mu * mu + 1e-10)).reshape(1, 1)


def _ffn_kernel(be_ref, xg_ref, w1_ref, w2_ref, ws_ref, yp_ref):
    del be_ref
    h = _dot_t(xg_ref[...], w1_ref[0])
    h = h * jax.nn.sigmoid(h)
    y = _dot_t(h, w2_ref[0])
    yp_ref[...] = y * ws_ref[...]


def _comb_kernel(x2_ref, g_ref, out_ref):
    out_ref[...] = x2_ref[...] + g_ref[:, 0] + g_ref[:, 1]


def _sc_gather(table, idx, n_rows, n_cols, chunk):
    info = plsc.get_sparse_core_info()
    nc = info.num_cores
    nw = nc * info.num_subcores
    rpw = n_rows // nw
    nchunks = rpw // chunk
    dt = table.dtype
    mesh = plsc.VectorSubcoreMesh(core_axis_name="c", subcore_axis_name="s")

    @functools.partial(
        pl.kernel,
        out_type=jax.ShapeDtypeStruct((n_rows, n_cols), dt),
        mesh=mesh,
        scratch_types=[
            pltpu.VMEM((rpw,), jnp.int32),
            pltpu.VMEM((chunk, n_cols), dt),
            pltpu.VMEM((chunk, n_cols), dt),
            pltpu.SemaphoreType.DMA,
            pltpu.SemaphoreType.DMA,
            pltpu.SemaphoreType.DMA,
            pltpu.SemaphoreType.DMA,
        ],
    )
    def gk(table_hbm, idx_hbm, out_hbm, idx_v, rows_a, rows_b, gs_a, gs_b,
           os_a, os_b):
        wid = jax.lax.axis_index("s") * nc + jax.lax.axis_index("c")
        base = wid * rpw
        pltpu.sync_copy(idx_hbm.at[pl.ds(base, rpw)], idx_v)
        bufs = (rows_a, rows_b)
        gsems = (gs_a, gs_b)
        osems = (os_a, os_b)

        def gstart(c):
            b = c % 2
            return pltpu.async_copy(
                table_hbm.at[idx_v.at[pl.ds(c * chunk, chunk)]], bufs[b],
                gsems[b])

        def ostart(c):
            b = c % 2
            return pltpu.async_copy(
                bufs[b], out_hbm.at[pl.ds(base + c * chunk, chunk)], osems[b])

        gh = [None] * nchunks
        oh = [None] * nchunks
        gh[0] = gstart(0)
        for c in range(nchunks):
            gh[c].wait()
            if c + 1 < nchunks:
                if c >= 1:
                    oh[c - 1].wait()
                gh[c + 1] = gstart(c + 1)
            oh[c] = ostart(c)
        oh[nchunks - 1].wait()
        if nchunks >= 2:
            oh[nchunks - 2].wait()

    return gk(table, idx)


def kernel(x, attention_mask, norm1_w, qkv_w, out_w, norm2_w, gating_w, W1, W2):
    B, S, E = x.shape
    N = B * S
    H = 16
    D = E // H
    NE, FH, _ = W1.shape

    BM = 256
    BQ = 256
    BN = 512
    BT = 256
    N2 = 2 * N
    P = N2 + NE * BT
    NB = P // BT
    f32 = jnp.float32

    xf = x.reshape(N, E)
    n1 = norm1_w.reshape(1, E)
    n2 = norm2_w.reshape(1, E)

    nj = E // BN
    ni = N // BM
    q, k, v = pl.pallas_call(
        _qkv_kernel,
        grid=(nj, ni),
        in_specs=[
            pl.BlockSpec((BM, E), lambda j, i: (i, 0)),
            pl.BlockSpec((1, E), lambda j, i: (0, 0)),
            pl.BlockSpec((BN, E), lambda j, i: (j, 0)),
            pl.BlockSpec((BN, E), lambda j, i: (j + nj, 0)),
            pl.BlockSpec((BN, E), lambda j, i: (j + 2 * nj, 0)),
        ],
        out_specs=[
            pl.BlockSpec((BM, BN), lambda j, i: (i, j)),
            pl.BlockSpec((BM, BN), lambda j, i: (i, j)),
            pl.BlockSpec((BM, BN), lambda j, i: (i, j)),
        ],
        out_shape=[jax.ShapeDtypeStruct((N, E), f32)] * 3,
    )(xf, n1, qkv_w, qkv_w, qkv_w)

    q3 = q.reshape(B * H, S, D)
    k3 = k.reshape(B * H, S, D)
    v3 = v.reshape(B * H, S, D)

    o = pl.pallas_call(
        functools.partial(_attn_kernel, scale=1.0 / (D ** 0.5), bq=BQ),
        grid=(B * H, S // BQ),
        in_specs=[
            pl.BlockSpec((1, BQ, D), lambda b, i: (b, i, 0)),
            pl.BlockSpec((1, S, D), lambda b, i: (b, 0, 0)),
            pl.BlockSpec((1, S, D), lambda b, i: (b, 0, 0)),
        ],
        out_specs=pl.BlockSpec((1, BQ, D), lambda b, i: (b, i, 0)),
        out_shape=jax.ShapeDtypeStruct((B * H, S, D), f32),
    )(q3, k3, v3)
    o2 = o.reshape(N, E)

    x2, h2 = pl.pallas_call(
        _proj_kernel,
        grid=(ni,),
        in_specs=[
            pl.BlockSpec((BM, E), lambda i: (i, 0)),
            pl.BlockSpec((BM, E), lambda i: (i, 0)),
            pl.BlockSpec((E, E), lambda i: (0, 0)),
            pl.BlockSpec((1, E), lambda i: (0, 0)),
        ],
        out_specs=[
            pl.BlockSpec((BM, E), lambda i: (i, 0)),
            pl.BlockSpec((BM, E), lambda i: (i, 0)),
        ],
        out_shape=[jax.ShapeDtypeStruct((N, E), f32)] * 2,
    )(o2, xf, out_w, n2)

    i1, i2, w1, w2, loss = pl.pallas_call(
        functools.partial(_gate_kernel, n_total=float(N), n_blocks=ni),
        grid=(ni,),
        in_specs=[
            pl.BlockSpec((BM, E), lambda i: (i, 0)),
            pl.BlockSpec((NE, E), lambda i: (0, 0)),
        ],
        out_specs=[
            pl.BlockSpec((BM, 1), lambda i: (i, 0)),
            pl.BlockSpec((BM, 1), lambda i: (i, 0)),
            pl.BlockSpec((BM, 1), lambda i: (i, 0)),
            pl.BlockSpec((BM, 1), lambda i: (i, 0)),
            pl.BlockSpec((1, 1), lambda i: (0, 0)),
        ],
        out_shape=[
            jax.ShapeDtypeStruct((N, 1), jnp.int32),
            jax.ShapeDtypeStruct((N, 1), jnp.int32),
            jax.ShapeDtypeStruct((N, 1), f32),
            jax.ShapeDtypeStruct((N, 1), f32),
            jax.ShapeDtypeStruct((1, 1), f32),
        ],
        scratch_shapes=[pltpu.VMEM((1, NE), f32)],
    )(h2, gating_w)

    e_flat = jnp.concatenate([i1, i2], axis=1).reshape(N2)
    w_flat = jnp.concatenate([w1, w2], axis=1).reshape(N2)
    oh = (e_flat[:, None] == jnp.arange(NE, dtype=jnp.int32)).astype(jnp.int32)
    csum = jnp.cumsum(oh, axis=0)
    counts = csum[-1]
    rank = jnp.take_along_axis(csum, e_flat[:, None], axis=1)[:, 0] - 1
    nblk = (counts + BT - 1) // BT
    poff = BT * jnp.concatenate(
        [jnp.zeros((1,), jnp.int32), jnp.cumsum(nblk)]).astype(jnp.int32)
    pos = poff[e_flat] + rank
    tok = (jnp.arange(P, dtype=jnp.int32) % N).at[pos].set(
        jnp.arange(N2, dtype=jnp.int32) // 2)
    wslot = jnp.zeros((P, 1), f32).at[pos].set(w_flat[:, None])
    block_expert = jnp.clip(
        jnp.searchsorted(poff[1:], jnp.arange(NB, dtype=jnp.int32) * BT,
                         side="right"),
        0, NE - 1).astype(jnp.int32)

    xg = _sc_gather(h2, tok, P, E, chunk=16)

    yp = pl.pallas_call(
        _ffn_kernel,
        grid_spec=pltpu.PrefetchScalarGridSpec(
            num_scalar_prefetch=1,
            grid=(NB,),
            in_specs=[
                pl.BlockSpec((BT, E), lambda b, be: (b, 0)),
                pl.BlockSpec((1, FH, E), lambda b, be: (be[b], 0, 0)),
                pl.BlockSpec((1, E, FH), lambda b, be: (be[b], 0, 0)),
                pl.BlockSpec((BT, 1), lambda b, be: (b, 0)),
            ],
            out_specs=pl.BlockSpec((BT, E), lambda b, be: (b, 0)),
        ),
        out_shape=jax.ShapeDtypeStruct((P, E), f32),
    )(block_expert, xg, W1, W2, wslot)

    g = _sc_gather(yp, pos, N2, E, chunk=16)
    g3 = g.reshape(N, 2, E)

    out = pl.pallas_call(
        _comb_kernel,
        grid=(ni,),
        in_specs=[
            pl.BlockSpec((BM, E), lambda i: (i, 0)),
            pl.BlockSpec((BM, 2, E), lambda i: (i, 0, 0)),
        ],
        out_specs=pl.BlockSpec((BM, E), lambda i: (i, 0)),
        out_shape=jax.ShapeDtypeStruct((N, E), f32),
    )(x2, g3)

    return out.reshape(B, S, E), loss[0, 0]


# speedup vs baseline: 1.1130x; 1.0205x over previous
import functools

import jax
import jax.numpy as jnp
from jax.experimental import pallas as pl
from jax.experimental.pallas import tpu as pltpu
from jax.experimental.pallas import tpu_sc as plsc

_EPS = 1.1920928955078125e-07


def _dot_t(a, b):
    return jax.lax.dot_general(a, b, (((1,), (1,)), ((), ())),
                               preferred_element_type=jnp.float32)


def _rms(x, w):
    ms = jnp.mean(x * x, axis=-1, keepdims=True)
    return x / jnp.sqrt(ms + _EPS) * w


def _qkv_kernel(x_ref, n1_ref, wq_ref, wk_ref, wv_ref, q_ref, k_ref, v_ref):
    xn = _rms(x_ref[...], n1_ref[...]).astype(jnp.bfloat16)
    q_ref[...] = _dot_t(xn, wq_ref[...]).astype(jnp.bfloat16)
    k_ref[...] = _dot_t(xn, wk_ref[...]).astype(jnp.bfloat16)
    v_ref[...] = _dot_t(xn, wv_ref[...]).astype(jnp.bfloat16)


def _attn_kernel(q_ref, k_ref, v_ref, o_ref, *, scale, bq):
    i = pl.program_id(1)
    q = q_ref[0]
    k = k_ref[0]
    v = v_ref[0]
    sc = _dot_t(q, k) * scale
    row = jax.lax.broadcasted_iota(jnp.int32, sc.shape, 0) + i * bq
    col = jax.lax.broadcasted_iota(jnp.int32, sc.shape, 1)
    sc = sc + jnp.where(col <= row, 1.0, -10000.0)
    m = jnp.max(sc, axis=-1, keepdims=True)
    p = jnp.exp(sc - m)
    p = (p / jnp.sum(p, axis=-1, keepdims=True)).astype(jnp.bfloat16)
    o_ref[0] = jnp.dot(p, v,
                       preferred_element_type=jnp.float32).astype(jnp.bfloat16)


def _proj_kernel(o_ref, res_ref, w_ref, n2_ref, x2_ref, h2_ref):
    x2 = _dot_t(o_ref[...], w_ref[...]) + res_ref[...]

    x2_ref[...] = x2
    h2_ref[...] = _rms(x2, n2_ref[...])


def _gate_kernel(h2_ref, gw_ref, i1_ref, i2_ref, w1_ref, w2_ref, loss_ref,
                 acc_ref, *, n_total, n_blocks):
    i = pl.program_id(0)
    g = _dot_t(h2_ref[...], gw_ref[...])
    m = jnp.max(g, axis=-1, keepdims=True)
    eg = jnp.exp(g - m)
    gate = eg / jnp.sum(eg, axis=-1, keepdims=True)
    i1 = jnp.argmax(gate, axis=-1)
    w1 = jnp.max(gate, axis=-1)
    lane = jax.lax.broadcasted_iota(jnp.int32, gate.shape, 1)
    oh1 = lane == i1[:, None]
    g2 = jnp.where(oh1, -jnp.inf, gate)
    i2 = jnp.argmax(g2, axis=-1)
    w2 = jnp.max(g2, axis=-1)
    i1_ref[...] = i1[:, None].astype(jnp.int32)
    i2_ref[...] = i2[:, None].astype(jnp.int32)
    w1_ref[...] = w1[:, None]
    w2_ref[...] = w2[:, None]
    psum = jnp.sum(gate, axis=0, keepdims=True)

    @pl.when(i == 0)
    def _():
        acc_ref[...] = psum

    @pl.when(i > 0)
    def _():
        acc_ref[...] += psum

    @pl.when(i == n_blocks - 1)
    def _():
        p = acc_ref[...] / n_total
        mu = jnp.mean(p)
        var = jnp.mean((p - mu) ** 2)
        loss_ref[...] = (var / (mu * mu + 1e-10)).reshape(1, 1)


def _ffn_kernel(be_ref, xg_ref, w1_ref, w2_ref, ws_ref, yp_ref):
    del be_ref
    h = _dot_t(xg_ref[...].astype(jnp.bfloat16), w1_ref[0])
    h = (h * jax.nn.sigmoid(h)).astype(jnp.bfloat16)
    y = _dot_t(h, w2_ref[0])
    yp_ref[...] = y * ws_ref[...]


def _comb_kernel(x2_ref, g_ref, out_ref):
    out_ref[...] = x2_ref[...] + g_ref[:, 0] + g_ref[:, 1]


def _sc_gather(table, idx, n_rows, n_cols, chunk):
    info = plsc.get_sparse_core_info()
    nc = info.num_cores
    nw = nc * info.num_subcores
    rpw = n_rows // nw
    nchunks = rpw // chunk
    dt = table.dtype
    mesh = plsc.VectorSubcoreMesh(core_axis_name="c", subcore_axis_name="s")

    @functools.partial(
        pl.kernel,
        out_type=jax.ShapeDtypeStruct((n_rows, n_cols), dt),
        mesh=mesh,
        scratch_types=[
            pltpu.VMEM((rpw,), jnp.int32),
            pltpu.VMEM((chunk, n_cols), dt),
            pltpu.VMEM((chunk, n_cols), dt),
            pltpu.SemaphoreType.DMA,
            pltpu.SemaphoreType.DMA,
            pltpu.SemaphoreType.DMA,
            pltpu.SemaphoreType.DMA,
        ],
    )
    def gk(table_hbm, idx_hbm, out_hbm, idx_v, rows_a, rows_b, gs_a, gs_b,
           os_a, os_b):
        wid = jax.lax.axis_index("s") * nc + jax.lax.axis_index("c")
        base = wid * rpw
        pltpu.sync_copy(idx_hbm.at[pl.ds(base, rpw)], idx_v)
        bufs = (rows_a, rows_b)
        gsems = (gs_a, gs_b)
        osems = (os_a, os_b)

        def gstart(c):
            b = c % 2
            return pltpu.async_copy(
                table_hbm.at[idx_v.at[pl.ds(c * chunk, chunk)]], bufs[b],
                gsems[b])

        def ostart(c):
            b = c % 2
            return pltpu.async_copy(
                bufs[b], out_hbm.at[pl.ds(base + c * chunk, chunk)], osems[b])

        gh = [None] * nchunks
        oh = [None] * nchunks
        gh[0] = gstart(0)
        for c in range(nchunks):
            gh[c].wait()
            if c + 1 < nchunks:
                if c >= 1:
                    oh[c - 1].wait()
                gh[c + 1] = gstart(c + 1)
            oh[c] = ostart(c)
        oh[nchunks - 1].wait()
        if nchunks >= 2:
            oh[nchunks - 2].wait()

    return gk(table, idx)


def kernel(x, attention_mask, norm1_w, qkv_w, out_w, norm2_w, gating_w, W1, W2):
    B, S, E = x.shape
    N = B * S
    H = 16
    D = E // H
    NE, FH, _ = W1.shape

    BM = 256
    BQ = 256
    BN = 512
    BT = 256
    N2 = 2 * N
    P = N2 + NE * BT
    NB = P // BT
    f32 = jnp.float32

    xf = x.reshape(N, E)
    n1 = norm1_w.reshape(1, E)
    n2 = norm2_w.reshape(1, E)
    qkv_b = qkv_w.astype(jnp.bfloat16)
    out_b = out_w.astype(jnp.bfloat16)
    W1b = W1.astype(jnp.bfloat16)
    W2b = W2.astype(jnp.bfloat16)

    nj = E // BN
    ni = N // BM
    q, k, v = pl.pallas_call(
        _qkv_kernel,
        grid=(nj, ni),
        in_specs=[
            pl.BlockSpec((BM, E), lambda j, i: (i, 0)),
            pl.BlockSpec((1, E), lambda j, i: (0, 0)),
            pl.BlockSpec((BN, E), lambda j, i: (j, 0)),
            pl.BlockSpec((BN, E), lambda j, i: (j + nj, 0)),
            pl.BlockSpec((BN, E), lambda j, i: (j + 2 * nj, 0)),
        ],
        out_specs=[
            pl.BlockSpec((BM, BN), lambda j, i: (i, j)),
            pl.BlockSpec((BM, BN), lambda j, i: (i, j)),
            pl.BlockSpec((BM, BN), lambda j, i: (i, j)),
        ],
        out_shape=[jax.ShapeDtypeStruct((N, E), jnp.bfloat16)] * 3,
    )(xf, n1, qkv_b, qkv_b, qkv_b)

    q3 = q.reshape(B * H, S, D)
    k3 = k.reshape(B * H, S, D)
    v3 = v.reshape(B * H, S, D)

    o = pl.pallas_call(
        functools.partial(_attn_kernel, scale=1.0 / (D ** 0.5), bq=BQ),
        grid=(B * H, S // BQ),
        in_specs=[
            pl.BlockSpec((1, BQ, D), lambda b, i: (b, i, 0)),
            pl.BlockSpec((1, S, D), lambda b, i: (b, 0, 0)),
            pl.BlockSpec((1, S, D), lambda b, i: (b, 0, 0)),
        ],
        out_specs=pl.BlockSpec((1, BQ, D), lambda b, i: (b, i, 0)),
        out_shape=jax.ShapeDtypeStruct((B * H, S, D), jnp.bfloat16),
    )(q3, k3, v3)
    o2 = o.reshape(N, E)

    x2, h2 = pl.pallas_call(
        _proj_kernel,
        grid=(ni,),
        in_specs=[
            pl.BlockSpec((BM, E), lambda i: (i, 0)),
            pl.BlockSpec((BM, E), lambda i: (i, 0)),
            pl.BlockSpec((E, E), lambda i: (0, 0)),
            pl.BlockSpec((1, E), lambda i: (0, 0)),
        ],
        out_specs=[
            pl.BlockSpec((BM, E), lambda i: (i, 0)),
            pl.BlockSpec((BM, E), lambda i: (i, 0)),
        ],
        out_shape=[jax.ShapeDtypeStruct((N, E), f32)] * 2,
    )(o2, xf, out_b, n2)

    i1, i2, w1, w2, loss = pl.pallas_call(
        functools.partial(_gate_kernel, n_total=float(N), n_blocks=ni),
        grid=(ni,),
        in_specs=[
            pl.BlockSpec((BM, E), lambda i: (i, 0)),
            pl.BlockSpec((NE, E), lambda i: (0, 0)),
        ],
        out_specs=[
            pl.BlockSpec((BM, 1), lambda i: (i, 0)),
            pl.BlockSpec((BM, 1), lambda i: (i, 0)),
            pl.BlockSpec((BM, 1), lambda i: (i, 0)),
            pl.BlockSpec((BM, 1), lambda i: (i, 0)),
            pl.BlockSpec((1, 1), lambda i: (0, 0)),
        ],
        out_shape=[
            jax.ShapeDtypeStruct((N, 1), jnp.int32),
            jax.ShapeDtypeStruct((N, 1), jnp.int32),
            jax.ShapeDtypeStruct((N, 1), f32),
            jax.ShapeDtypeStruct((N, 1), f32),
            jax.ShapeDtypeStruct((1, 1), f32),
        ],
        scratch_shapes=[pltpu.VMEM((1, NE), f32)],
    )(h2, gating_w)

    e_flat = jnp.concatenate([i1, i2], axis=1).reshape(N2)
    w_flat = jnp.concatenate([w1, w2], axis=1).reshape(N2)
    oh = (e_flat[:, None] == jnp.arange(NE, dtype=jnp.int32)).astype(jnp.int32)
    csum = jnp.cumsum(oh, axis=0)
    counts = csum[-1]
    rank = jnp.take_along_axis(csum, e_flat[:, None], axis=1)[:, 0] - 1
    nblk = (counts + BT - 1) // BT
    poff = BT * jnp.concatenate(
        [jnp.zeros((1,), jnp.int32), jnp.cumsum(nblk)]).astype(jnp.int32)
    pos = poff[e_flat] + rank
    tok = (jnp.arange(P, dtype=jnp.int32) % N).at[pos].set(
        jnp.arange(N2, dtype=jnp.int32) // 2)
    wslot = jnp.zeros((P, 1), f32).at[pos].set(w_flat[:, None])
    block_expert = jnp.clip(
        jnp.searchsorted(poff[1:], jnp.arange(NB, dtype=jnp.int32) * BT,
                         side="right"),
        0, NE - 1).astype(jnp.int32)

    xg = _sc_gather(h2, tok, P, E, chunk=16)

    yp = pl.pallas_call(
        _ffn_kernel,
        grid_spec=pltpu.PrefetchScalarGridSpec(
            num_scalar_prefetch=1,
            grid=(NB,),
            in_specs=[
                pl.BlockSpec((BT, E), lambda b, be: (b, 0)),
                pl.BlockSpec((1, FH, E), lambda b, be: (be[b], 0, 0)),
                pl.BlockSpec((1, E, FH), lambda b, be: (be[b], 0, 0)),
                pl.BlockSpec((BT, 1), lambda b, be: (b, 0)),
            ],
            out_specs=pl.BlockSpec((BT, E), lambda b, be: (b, 0)),
        ),
        out_shape=jax.ShapeDtypeStruct((P, E), f32),
    )(block_expert, xg, W1b, W2b, wslot)

    g = _sc_gather(yp, pos, N2, E, chunk=16)
    g3 = g.reshape(N, 2, E)

    out = pl.pallas_call(
        _comb_kernel,
        grid=(ni,),
        in_specs=[
            pl.BlockSpec((BM, E), lambda i: (i, 0)),
            pl.BlockSpec((BM, 2, E), lambda i: (i, 0, 0)),
        ],
        out_specs=pl.BlockSpec((BM, E), lambda i: (i, 0)),
        out_shape=jax.ShapeDtypeStruct((N, E), f32),
    )(x2, g3)

    return out.reshape(B, S, E), loss[0, 0]
